# baseline (speedup 1.0000x reference)
import jax
import jax.numpy as jnp
from jax.experimental import pallas as pl

D_MODEL = 1024
RANK = 128
N_INPUT = 8
N_PROCESS = 32
N_OUTPUT = 8
K = 4

TILE = 512


def _body(x_ref, wi_ref, idx_ref, wo_ref, win_ref, pn_ref, won_ref, out_ref):
    T = x_ref.shape[0]
    xb = x_ref[...]
    wi = wi_ref[...]

    P = jnp.dot(xb, win_ref[...], preferred_element_type=jnp.float32)
    P = P.reshape(T, N_INPUT, RANK)
    h = jnp.sum(P * wi[:, :, None], axis=1)

    pn = pn_ref[...]
    pn = pn * jax.lax.rsqrt(jnp.sum(pn * pn, axis=1, keepdims=True) + 1e-8)
    idx = idx_ref[...]
    iota = jax.lax.broadcasted_iota(jnp.int32, (T, N_PROCESS), 1)
    for i in range(K):
        oh = (idx[:, i : i + 1] == iota).astype(jnp.float32)
        v = jnp.dot(oh, pn, preferred_element_type=jnp.float32)
        h = h - 2.0 * v * jnp.sum(h * v, axis=1, keepdims=True)

    wo = wo_ref[...]
    hw = (wo[:, :, None] * h[:, None, :]).reshape(T, N_OUTPUT * RANK)
    out_ref[...] = jnp.dot(hw.astype(jnp.bfloat16), won_ref[...],
                           preferred_element_type=jnp.float32)


def kernel(x, input_weights, process_indices, output_weights,
           input_neurons, process_neurons, output_neurons):
    B, S, D = x.shape
    N = B * S
    xf = x.reshape(N, D).astype(jnp.bfloat16)
    wif = input_weights.reshape(N, N_INPUT)
    idxf = process_indices.reshape(N, K).astype(jnp.int32)
    wof = output_weights.reshape(N, N_OUTPUT)
    win_t = jnp.transpose(input_neurons, (1, 0, 2)).reshape(D, N_INPUT * RANK).astype(jnp.bfloat16)
    won_f = output_neurons.reshape(N_OUTPUT * RANK, D).astype(jnp.bfloat16)

    grid = (N // TILE,)
    out = pl.pallas_call(
        _body,
        grid=grid,
        in_specs=[
            pl.BlockSpec((TILE, D), lambda i: (i, 0)),
            pl.BlockSpec((TILE, N_INPUT), lambda i: (i, 0)),
            pl.BlockSpec((TILE, K), lambda i: (i, 0)),
            pl.BlockSpec((TILE, N_OUTPUT), lambda i: (i, 0)),
            pl.BlockSpec((D, N_INPUT * RANK), lambda i: (0, 0)),
            pl.BlockSpec((N_PROCESS, RANK), lambda i: (0, 0)),
            pl.BlockSpec((N_OUTPUT * RANK, D), lambda i: (0, 0)),
        ],
        out_specs=pl.BlockSpec((TILE, D), lambda i: (i, 0)),
        out_shape=jax.ShapeDtypeStruct((N, D), jnp.float32),
    )(xf, wif, idxf, wof, win_t, process_neurons, won_f)
    return out.reshape(B, S, D)


# speedup vs baseline: 1.2571x; 1.2571x over previous
import jax
import jax.numpy as jnp
from jax.experimental import pallas as pl

D_MODEL = 1024
RANK = 128
N_INPUT = 8
N_PROCESS = 32
N_OUTPUT = 8
K = 4

TILE = 512


def _body(x_ref, wi_ref, idx_ref, wo_ref, win_ref, pn_ref, won_ref,
          e_ref, g_ref, grep_ref, out_ref):
    T = x_ref.shape[0]
    f32 = jnp.float32
    bf16 = jnp.bfloat16

    P = jnp.dot(x_ref[...].astype(bf16), win_ref[...], preferred_element_type=f32)
    wiexp = jnp.dot(wi_ref[...], e_ref[...], preferred_element_type=f32)
    h = jnp.dot((P * wiexp).astype(bf16), g_ref[...], preferred_element_type=f32)

    pn = pn_ref[...]
    pnhat = pn * jax.lax.rsqrt(jnp.sum(pn * pn, axis=1, keepdims=True) + 1e-8)
    pnhat16 = pnhat.astype(bf16)
    pnhatT16 = pnhat.T.astype(bf16)
    idx = idx_ref[...]
    iota = jax.lax.broadcasted_iota(jnp.int32, (T, N_PROCESS), 1)
    for i in range(K):
        g = jnp.dot(h.astype(bf16), pnhatT16, preferred_element_type=f32)
        oh = idx[:, i : i + 1] == iota
        sel = jnp.where(oh, g * 2.0, 0.0)
        h = h - jnp.dot(sel.astype(bf16), pnhat16, preferred_element_type=f32)

    woexp = jnp.dot(wo_ref[...], e_ref[...], preferred_element_type=f32)
    hrep = jnp.dot(h.astype(bf16), grep_ref[...], preferred_element_type=f32)
    out_ref[...] = jnp.dot((hrep * woexp).astype(bf16), won_ref[...],
                           preferred_element_type=f32)


def kernel(x, input_weights, process_indices, output_weights,
           input_neurons, process_neurons, output_neurons):
    B, S, D = x.shape
    N = B * S
    xf = x.reshape(N, D)
    wif = input_weights.reshape(N, N_INPUT)
    idxf = process_indices.reshape(N, K).astype(jnp.int32)
    wof = output_weights.reshape(N, N_OUTPUT)
    win_t = jnp.transpose(input_neurons, (1, 0, 2)).reshape(D, N_INPUT * RANK).astype(jnp.bfloat16)
    won_f = output_neurons.reshape(N_OUTPUT * RANK, D).astype(jnp.bfloat16)
    eye_r = jnp.eye(RANK, dtype=jnp.bfloat16)
    e_mat = jnp.repeat(jnp.eye(N_INPUT, dtype=jnp.float32), RANK, axis=1)
    g_mat = jnp.tile(eye_r, (N_INPUT, 1))
    grep_mat = jnp.tile(eye_r, (1, N_OUTPUT))

    grid = (N // TILE,)
    out = pl.pallas_call(
        _body,
        grid=grid,
        in_specs=[
            pl.BlockSpec((TILE, D), lambda i: (i, 0)),
            pl.BlockSpec((TILE, N_INPUT), lambda i: (i, 0)),
            pl.BlockSpec((TILE, K), lambda i: (i, 0)),
            pl.BlockSpec((TILE, N_OUTPUT), lambda i: (i, 0)),
            pl.BlockSpec((D, N_INPUT * RANK), lambda i: (0, 0)),
            pl.BlockSpec((N_PROCESS, RANK), lambda i: (0, 0)),
            pl.BlockSpec((N_OUTPUT * RANK, D), lambda i: (0, 0)),
            pl.BlockSpec((N_INPUT, N_INPUT * RANK), lambda i: (0, 0)),
            pl.BlockSpec((N_INPUT * RANK, RANK), lambda i: (0, 0)),
            pl.BlockSpec((RANK, N_OUTPUT * RANK), lambda i: (0, 0)),
        ],
        out_specs=pl.BlockSpec((TILE, D), lambda i: (i, 0)),
        out_shape=jax.ShapeDtypeStruct((N, D), jnp.float32),
    )(xf, wif, idxf, wof, win_t, process_neurons, won_f, e_mat, g_mat, grep_mat)
    return out.reshape(B, S, D)
